# baseline (speedup 1.0000x reference)
import functools

import jax
import jax.numpy as jnp
from jax import lax
from jax.experimental import pallas as pl
from jax.experimental.pallas import tpu as pltpu
from jax.experimental.pallas import tpu_sc as plsc

F32 = jnp.float32
N_TOK = 1024 * 50
NW = 32
TW = N_TOK // NW
CH = 80
NCH = TW // CH

_OFF = {
    'test': (0, 21), 'tag': (21, 21), 'question': (42, 21), 'interaction': (63, 21),
    'duration': (84, 1), 'test_group_one': (85, 115), 'test_group_two': (200, 115),
    'serial': (315, 100), 'scalars_a': (415, 9), 'tag_group_one': (424, 50),
    'tag_group_two': (474, 50), 'time_for_solve': (524, 1), 'guess0': (525, 90),
    'day_of_week': (615, 16), 'scalars_b': (631, 25),
}

_HI = lax.Precision.HIGHEST


def _dgT(x, w):
    return lax.dot_general(x, w, (((1,), (1,)), ((), ())),
                           preferred_element_type=F32, precision=_HI)


def _prep_body(wt, wtag, wq, wtg, wser, wint, g0, g1, g2, g3, g4, g5, g6, g7,
               g8, wday, wc, pt0, pt1, pt2, pt3, pt4, pt5, pm):
    W = wc[...]
    pt0[...] = _dgT(wt[...], W[:, 0:21])
    pt1[...] = _dgT(wtag[...], W[:, 21:42])
    pt2[...] = _dgT(wq[...], W[:, 42:63])
    pt3[...] = _dgT(wtg[...], W[:, 85:200])
    pt4[...] = _dgT(wtg[...], W[:, 200:315])
    pt5[...] = _dgT(wser[...], W[:, 315:415])
    rows = [_dgT(wint[...], W[:, 63:84])]
    for t, g in enumerate((g0, g1, g2, g3, g4, g5, g6, g7, g8)):
        rows.append(_dgT(g[...], W[:, 525 + 10 * t:535 + 10 * t]))
    rows.append(_dgT(wday[...], W[:, 615:631]))
    pm[...] = jnp.concatenate(rows, axis=0)


def _prep(wt, wtag, wq, wtg, wser, wint, gs, wday, wc, interpret=False):
    out_shape = [
        jax.ShapeDtypeStruct((1539, 64), F32),
        jax.ShapeDtypeStruct((914, 64), F32),
        jax.ShapeDtypeStruct((9456, 64), F32),
        jax.ShapeDtypeStruct((1001, 64), F32),
        jax.ShapeDtypeStruct((1001, 64), F32),
        jax.ShapeDtypeStruct((1001, 64), F32),
        jax.ShapeDtypeStruct((37, 64), F32),
    ]
    return pl.pallas_call(_prep_body, out_shape=out_shape, interpret=interpret)(
        wt, wtag, wq, wtg, wser, wint, *gs, wday, wc)


def _sc_body(pt0, pt1, pt2, pt3, pt4, pt5, wg1f, wg2f, idx3,
             acc_out, g1_out, g2_out,
             idxv, rb0, rb1, rb2, rb3, rb4, rb5, bb1, bb2, accv,
             s0, s1, s2, s3, s4, s5, s6, s7):
    wid = lax.axis_index("s") * 2 + lax.axis_index("c")

    def chunk_body(k, carry):
        cg = wid * NCH + k
        base = wid * TW + k * CH
        pltpu.sync_copy(idx3.at[cg], idxv)
        cps = [
            pltpu.async_copy(pt0.at[idxv.at[0]], rb0, s0),
            pltpu.async_copy(pt1.at[idxv.at[1]], rb1, s1),
            pltpu.async_copy(pt2.at[idxv.at[2]], rb2, s2),
            pltpu.async_copy(pt3.at[idxv.at[3]], rb3, s3),
            pltpu.async_copy(pt4.at[idxv.at[4]], rb4, s4),
            pltpu.async_copy(pt5.at[idxv.at[5]], rb5, s5),
        ]
        def win_body(g, c2):
            r1v = idxv[6, pl.ds(g * 16, 16)]
            r2v = idxv[7, pl.ds(g * 16, 16)]
            for l in range(16):
                j = g * 16 + l
                pltpu.async_copy(wg1f.at[r1v[l]], bb1.at[j], s6)
                pltpu.async_copy(wg2f.at[r2v[l]], bb2.at[j], s7)
            return c2
        lax.fori_loop(0, CH // 16, win_body, 0)

        for cp in cps:
            cp.wait()

        def row_body(r, c2):
            for c in range(4):
                sl = pl.ds(c * 16, 16)
                accv[r, sl] = (rb0[r, sl] + rb1[r, sl] + rb2[r, sl]
                               + rb3[r, sl] + rb4[r, sl] + rb5[r, sl])
            return c2
        lax.fori_loop(0, CH, row_body, 0)

        def drain_body(j, c2):
            pltpu.make_async_copy(wg1f.at[0], bb1.at[j], s6).wait()
            pltpu.make_async_copy(wg2f.at[0], bb2.at[j], s7).wait()
            return c2
        lax.fori_loop(0, CH, drain_body, 0)

        pltpu.sync_copy(accv, acc_out.at[pl.ds(base, CH)])
        pltpu.sync_copy(bb1, g1_out.at[pl.ds(base, CH)])
        pltpu.sync_copy(bb2, g2_out.at[pl.ds(base, CH)])
        return carry

    lax.fori_loop(0, NCH, chunk_body, 0)


def _sc_gather(pts, wg1f, wg2f, idx3):
    mesh = plsc.VectorSubcoreMesh(core_axis_name="c", subcore_axis_name="s")
    out_type = [
        jax.ShapeDtypeStruct((N_TOK, 64), F32),
        jax.ShapeDtypeStruct((N_TOK, 50), F32),
        jax.ShapeDtypeStruct((N_TOK, 50), F32),
    ]
    scratch = [
        pltpu.VMEM((8, CH), jnp.int32),
        pltpu.VMEM((CH, 64), F32), pltpu.VMEM((CH, 64), F32),
        pltpu.VMEM((CH, 64), F32), pltpu.VMEM((CH, 64), F32),
        pltpu.VMEM((CH, 64), F32), pltpu.VMEM((CH, 64), F32),
        pltpu.VMEM((CH, 50), F32), pltpu.VMEM((CH, 50), F32),
        pltpu.VMEM((CH, 64), F32),
    ] + [pltpu.SemaphoreType.DMA] * 8
    run = pl.kernel(_sc_body, out_type=out_type, mesh=mesh,
                    scratch_types=scratch,
                    compiler_params=pltpu.CompilerParams(
                        use_tc_tiling_on_sc=False))
    return run(*pts, wg1f, wg2f, idx3)


_BLK = 2048
_BASES = tuple([0] + [3 + 3 * t for t in range(9)] + [30])


def _final_body(acc, win1, win2, s, idxm, wc, pm, bvec, out):
    W = wc[...]
    x = acc[...]
    x = x + _dgT(win1[...], W[:, 424:474])
    x = x + _dgT(win2[...], W[:, 474:524])
    wcs = jnp.concatenate([W[:, 84:85], W[:, 415:424], W[:, 524:525],
                           W[:, 631:656]], axis=1)
    x = x + _dgT(s[...], wcs)
    gi = idxm[...]
    iot = lax.broadcasted_iota(jnp.int32, (_BLK, 37), 1)
    oh = (iot == gi[:, 0:1]).astype(F32)
    for t in range(1, 11):
        oh = oh + (iot == gi[:, t:t + 1]).astype(F32)
    x = x + lax.dot_general(oh, pm[...], (((1,), (0,)), ((), ())),
                            preferred_element_type=F32, precision=_HI)
    out[...] = x + bvec[...]


def _final(acc, win1, win2, s, idxm, wc, pm, b2, interpret=False):
    nblk = N_TOK // _BLK
    bs = lambda shp: pl.BlockSpec(shp, lambda i: (i, 0))
    full = lambda shp: pl.BlockSpec(shp, lambda i: (0, 0))
    return pl.pallas_call(
        _final_body,
        grid=(nblk,),
        in_specs=[bs((_BLK, 64)), bs((_BLK, 50)), bs((_BLK, 50)),
                  bs((_BLK, 36)), bs((_BLK, 11)),
                  full((64, 656)), full((37, 64)), full((1, 64))],
        out_specs=bs((_BLK, 64)),
        out_shape=jax.ShapeDtypeStruct((N_TOK, 64), F32),
        interpret=interpret,
    )(acc, win1, win2, s, idxm, wc, pm, b2)


_SCALARS = ('duration', 'solved_count', 'correct_before', 'wrong_before',
            'same_tag_solved_count', 'same_tag_correct_before',
            'same_tag_wrong_before', 'item_correct_percent',
            'user_correct_percent', 'current_correct_count', 'time_for_solve',
            'zero', 'user_ability', 'day_correct_percent',
            'correct_percent_group_one', 'correct_percent_group_two',
            'correct_percent_serial', 'duration_user', 'user_mode_hour',
            'hour', 'year', 'user_mode_year', 'test_min_year',
            'test_mode_year', 'test_max_year', 'item_min_year',
            'item_mode_year', 'item_max_year', 'user_max_year',
            'user_min_year', 'user_period_year', 'test_count', 'item_count',
            'item_difficulty', 'time_diff', 'user_solve_count')


def kernel(test, question, tag, interaction, test_group_one, test_group_two, serial, tag_group_one, tag_group_two, guess_yn, guess_yn_user, guess_yn_test, guess_yn_serial, guess_yn_assessment, guess_yn_tag, guess_yn_day, guess_yn_group_one, guess_yn_group_two, day_of_week, correct, mask, duration, startTime, elapsedTime, solved_count, correct_before, wrong_before, same_tag_solved_count, same_tag_correct_before, same_tag_wrong_before, item_correct_percent, user_correct_percent, current_correct_count, time_for_solve, correct_percent_group_one, correct_percent_group_two, correct_percent_serial, duration_user, item_difficulty, zero, user_ability, day_correct_percent, user_mode_hour, hour, year, user_mode_year, test_min_year, test_mode_year, test_max_year, item_min_year, item_mode_year, item_max_year, user_max_year, user_min_year, user_period_year, test_count, item_count, time_diff, user_solve_count, W_interaction, W_test, W_question, W_tag, W_test_group_one, W_serial, W_tag_group_one, W_tag_group_two, W_guess, W_guess_user, W_guess_test, W_guess_serial, W_guess_assessment, W_guess_tag, W_guess_day, W_guess_group_one, W_guess_group_two, W_day, comb_W, comb_b):
    env = dict(locals())
    B = test.shape[0]

    gs = (W_guess, W_guess_user, W_guess_test, W_guess_serial,
          W_guess_assessment, W_guess_tag, W_guess_day, W_guess_group_one,
          W_guess_group_two)
    pt0, pt1, pt2, pt3, pt4, pt5, pm = _prep(
        W_test, W_tag, W_question, W_test_group_one, W_serial, W_interaction,
        gs, W_day, comb_W)

    idx_sc = jnp.stack([test, tag, question, test_group_one, test_group_two,
                        serial, tag_group_one, tag_group_two])
    idx3 = idx_sc.reshape(8, N_TOK // CH, CH).transpose(1, 0, 2)
    acc, win1, win2 = _sc_gather((pt0, pt1, pt2, pt3, pt4, pt5),
                                 W_tag_group_one, W_tag_group_two, idx3)

    s = jnp.stack([env[n] for n in _SCALARS], axis=-1).reshape(N_TOK, 36)
    minis = (interaction, guess_yn, guess_yn_user, guess_yn_test,
             guess_yn_serial, guess_yn_assessment, guess_yn_tag, guess_yn_day,
             guess_yn_group_one, guess_yn_group_two, day_of_week)
    idxm = (jnp.stack(minis, axis=-1).reshape(N_TOK, 11)
            + jnp.array(_BASES, jnp.int32))
    X = _final(acc, win1, win2, s, idxm, comb_W, pm,
               comb_b.reshape(1, 64))
    return X.reshape(B, 50, 64), B


# speedup vs baseline: 1.1474x; 1.1474x over previous
import functools

import jax
import jax.numpy as jnp
from jax import lax
from jax.experimental import pallas as pl
from jax.experimental.pallas import tpu as pltpu
from jax.experimental.pallas import tpu_sc as plsc

F32 = jnp.float32
N_TOK = 1024 * 50
NW = 32
TW = N_TOK // NW
CH = 80
NCH = TW // CH

_OFF = {
    'test': (0, 21), 'tag': (21, 21), 'question': (42, 21), 'interaction': (63, 21),
    'duration': (84, 1), 'test_group_one': (85, 115), 'test_group_two': (200, 115),
    'serial': (315, 100), 'scalars_a': (415, 9), 'tag_group_one': (424, 50),
    'tag_group_two': (474, 50), 'time_for_solve': (524, 1), 'guess0': (525, 90),
    'day_of_week': (615, 16), 'scalars_b': (631, 25),
}

_HI = lax.Precision.HIGHEST


def _dgT(x, w):
    return lax.dot_general(x, w, (((1,), (1,)), ((), ())),
                           preferred_element_type=F32, precision=_HI)


def _prep_body(wt, wtag, wq, wtg, wser, wint, g0, g1, g2, g3, g4, g5, g6, g7,
               g8, wday, wc, pt0, pt1, pt2, pt3, pt4, pt5, pm):
    W = wc[...]
    pt0[...] = _dgT(wt[...], W[:, 0:21])
    pt1[...] = _dgT(wtag[...], W[:, 21:42])
    pt2[...] = _dgT(wq[...], W[:, 42:63])
    pt3[...] = _dgT(wtg[...], W[:, 85:200])
    pt4[...] = _dgT(wtg[...], W[:, 200:315])
    pt5[...] = _dgT(wser[...], W[:, 315:415])
    rows = [_dgT(wint[...], W[:, 63:84])]
    for t, g in enumerate((g0, g1, g2, g3, g4, g5, g6, g7, g8)):
        rows.append(_dgT(g[...], W[:, 525 + 10 * t:535 + 10 * t]))
    rows.append(_dgT(wday[...], W[:, 615:631]))
    pm[...] = jnp.concatenate(rows, axis=0)


def _prep(wt, wtag, wq, wtg, wser, wint, gs, wday, wc, interpret=False):
    out_shape = [
        jax.ShapeDtypeStruct((1539, 64), F32),
        jax.ShapeDtypeStruct((914, 64), F32),
        jax.ShapeDtypeStruct((9456, 64), F32),
        jax.ShapeDtypeStruct((1001, 64), F32),
        jax.ShapeDtypeStruct((1001, 64), F32),
        jax.ShapeDtypeStruct((1001, 64), F32),
        jax.ShapeDtypeStruct((37, 64), F32),
    ]
    return pl.pallas_call(_prep_body, out_shape=out_shape, interpret=interpret)(
        wt, wtag, wq, wtg, wser, wint, *gs, wday, wc)


_RB = 4096
_NBLK_RE = -(-913001 // _RB)


def _repack_body(t, out):
    out[...] = jnp.concatenate(
        [t[...], jnp.zeros((t.shape[0], 14), F32)], axis=1)


def _repack(t):
    return pl.pallas_call(
        _repack_body,
        grid=(_NBLK_RE,),
        in_specs=[pl.BlockSpec((_RB, 50), lambda i: (i, 0))],
        out_specs=pl.BlockSpec((_RB, 64), lambda i: (i, 0)),
        out_shape=jax.ShapeDtypeStruct((913001, 64), F32),
    )(t)


def _sc_body(pt0, pt1, pt2, pt3, pt4, pt5, wg1f, wg2f, idx3,
             acc_out, g1_out, g2_out,
             idxv, rb0, rb1, rb2, rb3, rb4, rb5, bb1, bb2, accv,
             s0, s1, s2, s3, s4, s5, s6, s7):
    wid = lax.axis_index("s") * 2 + lax.axis_index("c")

    def chunk_body(k, carry):
        cg = wid * NCH + k
        base = wid * TW + k * CH
        pltpu.sync_copy(idx3.at[cg], idxv)
        cps = [
            pltpu.async_copy(pt0.at[idxv.at[0]], rb0, s0),
            pltpu.async_copy(pt1.at[idxv.at[1]], rb1, s1),
            pltpu.async_copy(pt2.at[idxv.at[2]], rb2, s2),
            pltpu.async_copy(pt3.at[idxv.at[3]], rb3, s3),
            pltpu.async_copy(pt4.at[idxv.at[4]], rb4, s4),
            pltpu.async_copy(pt5.at[idxv.at[5]], rb5, s5),
        ]
        cps.append(pltpu.async_copy(wg1f.at[idxv.at[6]], bb1, s6))
        cps.append(pltpu.async_copy(wg2f.at[idxv.at[7]], bb2, s7))

        for cp in cps:
            cp.wait()

        def row_body(r, c2):
            for c in range(4):
                sl = pl.ds(c * 16, 16)
                accv[r, sl] = (rb0[r, sl] + rb1[r, sl] + rb2[r, sl]
                               + rb3[r, sl] + rb4[r, sl] + rb5[r, sl])
            return c2
        lax.fori_loop(0, CH, row_body, 0)

        pltpu.sync_copy(accv, acc_out.at[pl.ds(base, CH)])
        pltpu.sync_copy(bb1, g1_out.at[pl.ds(base, CH)])
        pltpu.sync_copy(bb2, g2_out.at[pl.ds(base, CH)])
        return carry

    lax.fori_loop(0, NCH, chunk_body, 0)


def _sc_gather(pts, wg1f, wg2f, idx3):
    mesh = plsc.VectorSubcoreMesh(core_axis_name="c", subcore_axis_name="s")
    out_type = [
        jax.ShapeDtypeStruct((N_TOK, 64), F32),
        jax.ShapeDtypeStruct((N_TOK, 64), F32),
        jax.ShapeDtypeStruct((N_TOK, 64), F32),
    ]
    scratch = [
        pltpu.VMEM((8, CH), jnp.int32),
        pltpu.VMEM((CH, 64), F32), pltpu.VMEM((CH, 64), F32),
        pltpu.VMEM((CH, 64), F32), pltpu.VMEM((CH, 64), F32),
        pltpu.VMEM((CH, 64), F32), pltpu.VMEM((CH, 64), F32),
        pltpu.VMEM((CH, 64), F32), pltpu.VMEM((CH, 64), F32),
        pltpu.VMEM((CH, 64), F32),
    ] + [pltpu.SemaphoreType.DMA] * 8
    run = pl.kernel(_sc_body, out_type=out_type, mesh=mesh,
                    scratch_types=scratch,
                    compiler_params=pltpu.CompilerParams(
                        use_tc_tiling_on_sc=False))
    return run(*pts, wg1f, wg2f, idx3)


_BLK = 2048
_BASES = tuple([0] + [3 + 3 * t for t in range(9)] + [30])


def _final_body(acc, win1, win2, s, idxm, wc, pm, bvec, out):
    W = wc[...]
    x = acc[...]
    x = x + _dgT(win1[...][:, 0:50], W[:, 424:474])
    x = x + _dgT(win2[...][:, 0:50], W[:, 474:524])
    wcs = jnp.concatenate([W[:, 84:85], W[:, 415:424], W[:, 524:525],
                           W[:, 631:656]], axis=1)
    x = x + _dgT(s[...], wcs)
    gi = idxm[...]
    iot = lax.broadcasted_iota(jnp.int32, (_BLK, 37), 1)
    oh = (iot == gi[:, 0:1]).astype(F32)
    for t in range(1, 11):
        oh = oh + (iot == gi[:, t:t + 1]).astype(F32)
    x = x + lax.dot_general(oh, pm[...], (((1,), (0,)), ((), ())),
                            preferred_element_type=F32, precision=_HI)
    out[...] = x + bvec[...]


def _final(acc, win1, win2, s, idxm, wc, pm, b2, interpret=False):
    nblk = N_TOK // _BLK
    bs = lambda shp: pl.BlockSpec(shp, lambda i: (i, 0))
    full = lambda shp: pl.BlockSpec(shp, lambda i: (0, 0))
    return pl.pallas_call(
        _final_body,
        grid=(nblk,),
        in_specs=[bs((_BLK, 64)), bs((_BLK, 64)), bs((_BLK, 64)),
                  bs((_BLK, 36)), bs((_BLK, 11)),
                  full((64, 656)), full((37, 64)), full((1, 64))],
        out_specs=bs((_BLK, 64)),
        out_shape=jax.ShapeDtypeStruct((N_TOK, 64), F32),
        interpret=interpret,
    )(acc, win1, win2, s, idxm, wc, pm, b2)


_SCALARS = ('duration', 'solved_count', 'correct_before', 'wrong_before',
            'same_tag_solved_count', 'same_tag_correct_before',
            'same_tag_wrong_before', 'item_correct_percent',
            'user_correct_percent', 'current_correct_count', 'time_for_solve',
            'zero', 'user_ability', 'day_correct_percent',
            'correct_percent_group_one', 'correct_percent_group_two',
            'correct_percent_serial', 'duration_user', 'user_mode_hour',
            'hour', 'year', 'user_mode_year', 'test_min_year',
            'test_mode_year', 'test_max_year', 'item_min_year',
            'item_mode_year', 'item_max_year', 'user_max_year',
            'user_min_year', 'user_period_year', 'test_count', 'item_count',
            'item_difficulty', 'time_diff', 'user_solve_count')


def kernel(test, question, tag, interaction, test_group_one, test_group_two, serial, tag_group_one, tag_group_two, guess_yn, guess_yn_user, guess_yn_test, guess_yn_serial, guess_yn_assessment, guess_yn_tag, guess_yn_day, guess_yn_group_one, guess_yn_group_two, day_of_week, correct, mask, duration, startTime, elapsedTime, solved_count, correct_before, wrong_before, same_tag_solved_count, same_tag_correct_before, same_tag_wrong_before, item_correct_percent, user_correct_percent, current_correct_count, time_for_solve, correct_percent_group_one, correct_percent_group_two, correct_percent_serial, duration_user, item_difficulty, zero, user_ability, day_correct_percent, user_mode_hour, hour, year, user_mode_year, test_min_year, test_mode_year, test_max_year, item_min_year, item_mode_year, item_max_year, user_max_year, user_min_year, user_period_year, test_count, item_count, time_diff, user_solve_count, W_interaction, W_test, W_question, W_tag, W_test_group_one, W_serial, W_tag_group_one, W_tag_group_two, W_guess, W_guess_user, W_guess_test, W_guess_serial, W_guess_assessment, W_guess_tag, W_guess_day, W_guess_group_one, W_guess_group_two, W_day, comb_W, comb_b):
    env = dict(locals())
    B = test.shape[0]

    gs = (W_guess, W_guess_user, W_guess_test, W_guess_serial,
          W_guess_assessment, W_guess_tag, W_guess_day, W_guess_group_one,
          W_guess_group_two)
    pt0, pt1, pt2, pt3, pt4, pt5, pm = _prep(
        W_test, W_tag, W_question, W_test_group_one, W_serial, W_interaction,
        gs, W_day, comb_W)

    idx_sc = jnp.stack([test, tag, question, test_group_one, test_group_two,
                        serial, tag_group_one, tag_group_two])
    idx3 = idx_sc.reshape(8, N_TOK // CH, CH).transpose(1, 0, 2)
    acc, win1, win2 = _sc_gather((pt0, pt1, pt2, pt3, pt4, pt5),
                                 _repack(W_tag_group_one),
                                 _repack(W_tag_group_two), idx3)

    s = jnp.stack([env[n] for n in _SCALARS], axis=-1).reshape(N_TOK, 36)
    minis = (interaction, guess_yn, guess_yn_user, guess_yn_test,
             guess_yn_serial, guess_yn_assessment, guess_yn_tag, guess_yn_day,
             guess_yn_group_one, guess_yn_group_two, day_of_week)
    idxm = (jnp.stack(minis, axis=-1).reshape(N_TOK, 11)
            + jnp.array(_BASES, jnp.int32))
    X = _final(acc, win1, win2, s, idxm, comb_W, pm,
               comb_b.reshape(1, 64))
    return X.reshape(B, 50, 64), B


# speedup vs baseline: 1.6591x; 1.4459x over previous
import functools

import jax
import jax.numpy as jnp
from jax import lax
from jax.experimental import pallas as pl
from jax.experimental.pallas import tpu as pltpu
from jax.experimental.pallas import tpu_sc as plsc

F32 = jnp.float32
N_TOK = 1024 * 50
NW = 32
TW = N_TOK // NW
CH = 80
NCH = TW // CH

_OFF = {
    'test': (0, 21), 'tag': (21, 21), 'question': (42, 21), 'interaction': (63, 21),
    'duration': (84, 1), 'test_group_one': (85, 115), 'test_group_two': (200, 115),
    'serial': (315, 100), 'scalars_a': (415, 9), 'tag_group_one': (424, 50),
    'tag_group_two': (474, 50), 'time_for_solve': (524, 1), 'guess0': (525, 90),
    'day_of_week': (615, 16), 'scalars_b': (631, 25),
}

_HI = lax.Precision.HIGHEST


def _dgT(x, w):
    return lax.dot_general(x, w, (((1,), (1,)), ((), ())),
                           preferred_element_type=F32, precision=_HI)


def _prep_body(wt, wtag, wq, wtg, wser, wint, g0, g1, g2, g3, g4, g5, g6, g7,
               g8, wday, wc, pt0, pt1, pt2, pt3, pt4, pt5, pm):
    W = wc[...]
    pt0[...] = _dgT(wt[...], W[:, 0:21])
    pt1[...] = _dgT(wtag[...], W[:, 21:42])
    pt2[...] = _dgT(wq[...], W[:, 42:63])
    pt3[...] = _dgT(wtg[...], W[:, 85:200])
    pt4[...] = _dgT(wtg[...], W[:, 200:315])
    pt5[...] = _dgT(wser[...], W[:, 315:415])
    rows = [_dgT(wint[...], W[:, 63:84])]
    for t, g in enumerate((g0, g1, g2, g3, g4, g5, g6, g7, g8)):
        rows.append(_dgT(g[...], W[:, 525 + 10 * t:535 + 10 * t]))
    rows.append(_dgT(wday[...], W[:, 615:631]))
    pm[...] = jnp.concatenate(rows, axis=0)


def _prep(wt, wtag, wq, wtg, wser, wint, gs, wday, wc, interpret=False):
    out_shape = [
        jax.ShapeDtypeStruct((1539, 64), F32),
        jax.ShapeDtypeStruct((914, 64), F32),
        jax.ShapeDtypeStruct((9456, 64), F32),
        jax.ShapeDtypeStruct((1001, 64), F32),
        jax.ShapeDtypeStruct((1001, 64), F32),
        jax.ShapeDtypeStruct((1001, 64), F32),
        jax.ShapeDtypeStruct((37, 64), F32),
    ]
    return pl.pallas_call(_prep_body, out_shape=out_shape, interpret=interpret)(
        wt, wtag, wq, wtg, wser, wint, *gs, wday, wc)


_RB = 4096
_NBLK_RE = -(-913001 // _RB)


def _repack_body(t, out):
    out[...] = jnp.concatenate(
        [t[...], jnp.zeros((t.shape[0], 78), F32)], axis=1)


def _repack(t):
    return pl.pallas_call(
        _repack_body,
        grid=(_NBLK_RE,),
        in_specs=[pl.BlockSpec((_RB, 50), lambda i: (i, 0))],
        out_specs=pl.BlockSpec((_RB, 128), lambda i: (i, 0)),
        out_shape=jax.ShapeDtypeStruct((913001, 128), F32),
    )(t)


def _sc_body(pt0, pt1, pt2, pt3, pt4, pt5, wg1f, wg2f, idx3,
             acc_out, g1_out, g2_out,
             idxv, rb0, rb1, rb2, rb3, rb4, rb5, bb1, bb2, accv,
             s0, s1, s2, s3, s4, s5, s6, s7):
    wid = lax.axis_index("s") * 2 + lax.axis_index("c")

    def chunk_body(k, carry):
        cg = wid * NCH + k
        base = wid * TW + k * CH
        pltpu.sync_copy(idx3.at[cg], idxv)
        cps = [
            pltpu.async_copy(pt0.at[idxv.at[0]], rb0, s0),
            pltpu.async_copy(pt1.at[idxv.at[1]], rb1, s1),
            pltpu.async_copy(pt2.at[idxv.at[2]], rb2, s2),
            pltpu.async_copy(pt3.at[idxv.at[3]], rb3, s3),
            pltpu.async_copy(pt4.at[idxv.at[4]], rb4, s4),
            pltpu.async_copy(pt5.at[idxv.at[5]], rb5, s5),
        ]
        cps.append(pltpu.async_copy(wg1f.at[idxv.at[6]], bb1, s6))
        cps.append(pltpu.async_copy(wg2f.at[idxv.at[7]], bb2, s7))

        for cp in cps:
            cp.wait()

        def row_body(r, c2):
            for c in range(4):
                sl = pl.ds(c * 16, 16)
                accv[r, sl] = (rb0[r, sl] + rb1[r, sl] + rb2[r, sl]
                               + rb3[r, sl] + rb4[r, sl] + rb5[r, sl])
            return c2
        lax.fori_loop(0, CH, row_body, 0)

        pltpu.sync_copy(accv, acc_out.at[pl.ds(base, CH)])
        pltpu.sync_copy(bb1, g1_out.at[pl.ds(base, CH)])
        pltpu.sync_copy(bb2, g2_out.at[pl.ds(base, CH)])
        return carry

    lax.fori_loop(0, NCH, chunk_body, 0)


def _sc_gather(pts, wg1f, wg2f, idx3):
    mesh = plsc.VectorSubcoreMesh(core_axis_name="c", subcore_axis_name="s")
    out_type = [
        jax.ShapeDtypeStruct((N_TOK, 64), F32),
        jax.ShapeDtypeStruct((N_TOK, 128), F32),
        jax.ShapeDtypeStruct((N_TOK, 128), F32),
    ]
    scratch = [
        pltpu.VMEM((8, CH), jnp.int32),
        pltpu.VMEM((CH, 64), F32), pltpu.VMEM((CH, 64), F32),
        pltpu.VMEM((CH, 64), F32), pltpu.VMEM((CH, 64), F32),
        pltpu.VMEM((CH, 64), F32), pltpu.VMEM((CH, 64), F32),
        pltpu.VMEM((CH, 128), F32), pltpu.VMEM((CH, 128), F32),
        pltpu.VMEM((CH, 64), F32),
    ] + [pltpu.SemaphoreType.DMA] * 8
    run = pl.kernel(_sc_body, out_type=out_type, mesh=mesh,
                    scratch_types=scratch,
                    compiler_params=pltpu.CompilerParams(
                        use_tc_tiling_on_sc=False))
    return run(*pts, wg1f, wg2f, idx3)


_BLK = 2048
_BASES = tuple([0] + [3 + 3 * t for t in range(9)] + [30])


def _final_body(acc, win1, win2, s, idxm, wc, pm, bvec, out):
    W = wc[...]
    x = acc[...]
    x = x + _dgT(win1[...][:, 0:50], W[:, 424:474])
    x = x + _dgT(win2[...][:, 0:50], W[:, 474:524])
    wcs = jnp.concatenate([W[:, 84:85], W[:, 415:424], W[:, 524:525],
                           W[:, 631:656]], axis=1)
    x = x + _dgT(s[...], wcs)
    gi = idxm[...]
    iot = lax.broadcasted_iota(jnp.int32, (_BLK, 37), 1)
    oh = (iot == gi[:, 0:1]).astype(F32)
    for t in range(1, 11):
        oh = oh + (iot == gi[:, t:t + 1]).astype(F32)
    x = x + lax.dot_general(oh, pm[...], (((1,), (0,)), ((), ())),
                            preferred_element_type=F32, precision=_HI)
    out[...] = x + bvec[...]


def _final(acc, win1, win2, s, idxm, wc, pm, b2, interpret=False):
    nblk = N_TOK // _BLK
    bs = lambda shp: pl.BlockSpec(shp, lambda i: (i, 0))
    full = lambda shp: pl.BlockSpec(shp, lambda i: (0, 0))
    return pl.pallas_call(
        _final_body,
        grid=(nblk,),
        in_specs=[bs((_BLK, 64)), bs((_BLK, 128)), bs((_BLK, 128)),
                  bs((_BLK, 36)), bs((_BLK, 11)),
                  full((64, 656)), full((37, 64)), full((1, 64))],
        out_specs=bs((_BLK, 64)),
        out_shape=jax.ShapeDtypeStruct((N_TOK, 64), F32),
        interpret=interpret,
    )(acc, win1, win2, s, idxm, wc, pm, b2)


_SCALARS = ('duration', 'solved_count', 'correct_before', 'wrong_before',
            'same_tag_solved_count', 'same_tag_correct_before',
            'same_tag_wrong_before', 'item_correct_percent',
            'user_correct_percent', 'current_correct_count', 'time_for_solve',
            'zero', 'user_ability', 'day_correct_percent',
            'correct_percent_group_one', 'correct_percent_group_two',
            'correct_percent_serial', 'duration_user', 'user_mode_hour',
            'hour', 'year', 'user_mode_year', 'test_min_year',
            'test_mode_year', 'test_max_year', 'item_min_year',
            'item_mode_year', 'item_max_year', 'user_max_year',
            'user_min_year', 'user_period_year', 'test_count', 'item_count',
            'item_difficulty', 'time_diff', 'user_solve_count')


def kernel(test, question, tag, interaction, test_group_one, test_group_two, serial, tag_group_one, tag_group_two, guess_yn, guess_yn_user, guess_yn_test, guess_yn_serial, guess_yn_assessment, guess_yn_tag, guess_yn_day, guess_yn_group_one, guess_yn_group_two, day_of_week, correct, mask, duration, startTime, elapsedTime, solved_count, correct_before, wrong_before, same_tag_solved_count, same_tag_correct_before, same_tag_wrong_before, item_correct_percent, user_correct_percent, current_correct_count, time_for_solve, correct_percent_group_one, correct_percent_group_two, correct_percent_serial, duration_user, item_difficulty, zero, user_ability, day_correct_percent, user_mode_hour, hour, year, user_mode_year, test_min_year, test_mode_year, test_max_year, item_min_year, item_mode_year, item_max_year, user_max_year, user_min_year, user_period_year, test_count, item_count, time_diff, user_solve_count, W_interaction, W_test, W_question, W_tag, W_test_group_one, W_serial, W_tag_group_one, W_tag_group_two, W_guess, W_guess_user, W_guess_test, W_guess_serial, W_guess_assessment, W_guess_tag, W_guess_day, W_guess_group_one, W_guess_group_two, W_day, comb_W, comb_b):
    env = dict(locals())
    B = test.shape[0]

    gs = (W_guess, W_guess_user, W_guess_test, W_guess_serial,
          W_guess_assessment, W_guess_tag, W_guess_day, W_guess_group_one,
          W_guess_group_two)
    pt0, pt1, pt2, pt3, pt4, pt5, pm = _prep(
        W_test, W_tag, W_question, W_test_group_one, W_serial, W_interaction,
        gs, W_day, comb_W)

    idx_sc = jnp.stack([test, tag, question, test_group_one, test_group_two,
                        serial, tag_group_one, tag_group_two])
    idx3 = idx_sc.reshape(8, N_TOK // CH, CH).transpose(1, 0, 2)
    acc, win1, win2 = _sc_gather((pt0, pt1, pt2, pt3, pt4, pt5),
                                 _repack(W_tag_group_one),
                                 _repack(W_tag_group_two), idx3)

    s = jnp.stack([env[n] for n in _SCALARS], axis=-1).reshape(N_TOK, 36)
    minis = (interaction, guess_yn, guess_yn_user, guess_yn_test,
             guess_yn_serial, guess_yn_assessment, guess_yn_tag, guess_yn_day,
             guess_yn_group_one, guess_yn_group_two, day_of_week)
    idxm = (jnp.stack(minis, axis=-1).reshape(N_TOK, 11)
            + jnp.array(_BASES, jnp.int32))
    X = _final(acc, win1, win2, s, idxm, comb_W, pm,
               comb_b.reshape(1, 64))
    return X.reshape(B, 50, 64), B
